# baseline (speedup 1.0000x reference)
import functools

import jax
import jax.numpy as jnp
from jax import lax
from jax.experimental import pallas as pl
from jax.experimental.pallas import tpu as pltpu
from jax.experimental.pallas import tpu_sc as plsc

VOCAB = 1000000
D = 64
B = 4096
L = 200
BL = B * L

NC = 2
NS = 16
NW = NC * NS
BW = B // NW
CH_L = 4
NCH = L // CH_L
RPC = CH_L * BW
NK = D // 16
TC_ = 129


def _rsqrt(x):
    i = plsc.bitcast(x, jnp.int32)
    i = 0x5F3759DF - lax.shift_right_arithmetic(i, 1)
    y = plsc.bitcast(i, jnp.float32)
    for _ in range(2):
        y = y * (1.5 - 0.5 * x * y * y)
    return y


def _splat(v, lane):
    return jnp.take_along_axis(v, jnp.full((16,), lane, jnp.int32), axis=0)


def _sc_body(tokT_hbm, table_hbm, pos_hbm, gb_hbm, bb_hbm, out_hbm,
             idx_v, rows_v, pos_v, tbuf, gb_v, bb_v,
             sem_g0, sem_g1, sem_t0, sem_t1):
    wid = lax.axis_index("s") * NC + lax.axis_index("c")
    pltpu.sync_copy(pos_hbm, pos_v)
    pltpu.sync_copy(gb_hbm, gb_v)
    pltpu.sync_copy(bb_hbm, bb_v)
    gvec = [gb_v[pl.ds(k * 16, 16)] for k in range(NK)]
    bvec = [bb_v[pl.ds(k * 16, 16)] for k in range(NK)]
    lane = lax.iota(jnp.int32, 16)
    didx = [lane + k * 16 for k in range(NK)]
    gsems = [sem_g0, sem_g1]
    tsems = [sem_t0, sem_t1]

    def fetch_idx(buf, ci):
        for li in range(CH_L):
            pltpu.sync_copy(
                tokT_hbm.at[ci * CH_L + li, pl.ds(wid * BW, BW)],
                idx_v.at[buf, pl.ds(li * BW, BW)])

    def fire_gathers(buf, sem):
        for li in range(CH_L):
            pltpu.async_copy(
                table_hbm.at[idx_v.at[buf, pl.ds(li * BW, BW)]],
                rows_v.at[buf, pl.ds(li * BW, BW)], sem)

    def drain_gathers(buf, sem):
        for li in range(CH_L):
            pltpu.make_async_copy(
                table_hbm.at[idx_v.at[buf, pl.ds(li * BW, BW)]],
                rows_v.at[buf, pl.ds(li * BW, BW)], sem).wait()

    def drain_tiles(tb, lprev):
        for dt in range(8):
            pltpu.make_async_copy(
                tbuf.at[tb, pl.ds(dt * 8, 8), pl.ds(0, 128)],
                out_hbm.at[lprev, dt, wid], tsems[tb]).wait()

    def fire_tiles(tb, l):
        for dt in range(8):
            pltpu.async_copy(
                tbuf.at[tb, pl.ds(dt * 8, 8), pl.ds(0, 128)],
                out_hbm.at[l, dt, wid], tsems[tb])

    def do_pos(buf, l, lofs, tb, guard):
        @pl.when(guard)
        def _():
            drain_tiles(tb, l - 2)

        pk = [pos_v[l, pl.ds(k * 16, 16)] for k in range(NK)]

        def gbody(g, gcarry):
            base = lofs + g * 16
            tok_v = idx_v[buf, pl.ds(base, 16)]
            maskf = jnp.where(tok_v != 0, 1.0, 0.0).astype(jnp.float32)
            for r in range(16):
                row = base + r
                s = [rows_v[buf, row, pl.ds(k * 16, 16)] + pk[k]
                     for k in range(NK)]
                part = (s[0] + s[1]) + (s[2] + s[3])
                tot = _splat(plsc.cumsum(part), 15)
                sq = (s[0] * s[0] + s[1] * s[1]) + (s[2] * s[2] + s[3] * s[3])
                tot2 = _splat(plsc.cumsum(sq), 15)
                mu = tot * (1.0 / D)
                var = tot2 * (1.0 / D) - mu * mu + 1e-5
                rs = _rsqrt(var)
                m = _splat(maskf, r)
                a = rs * m
                bco = (0.0 - mu * rs) * m
                col = jnp.full((16,), g * 16 + r, jnp.int32)
                for k in range(NK):
                    o = (s[k] * a + bco) * gvec[k] + bvec[k] * m
                    plsc.store_scatter(tbuf.at[tb], [didx[k], col], o)
            return gcarry

        lax.fori_loop(0, BW // 16, gbody, 0)
        fire_tiles(tb, l)

    def compute_chunk(buf, ci):
        def lp_body(lp, carry):
            lbase = ci * CH_L + lp * 2
            do_pos(buf, lbase, lp * 2 * BW, 0, ci * 2 + lp > 0)
            do_pos(buf, lbase + 1, (lp * 2 + 1) * BW, 1, ci * 2 + lp > 0)
            return carry

        lax.fori_loop(0, CH_L // 2, lp_body, 0)

    fetch_idx(0, 0)
    fire_gathers(0, gsems[0])

    def half_body(b, ci):
        nb = 1 - b

        @pl.when(ci < NCH - 1)
        def _prefetch():
            fetch_idx(nb, ci + 1)
            fire_gathers(nb, gsems[nb])

        drain_gathers(b, gsems[b])
        compute_chunk(b, ci)

    def pair_body(cp, carry):
        half_body(0, cp * 2)
        half_body(1, cp * 2 + 1)
        return carry

    lax.fori_loop(0, NCH // 2, pair_body, 0)
    drain_tiles(0, L - 2)
    drain_tiles(1, L - 1)


def kernel(tokens, tok_table, pos_table, gamma, beta):
    tokT = tokens.T.astype(jnp.int32)

    sc = functools.partial(
        pl.kernel,
        mesh=plsc.VectorSubcoreMesh(core_axis_name="c", subcore_axis_name="s"),
        out_type=jax.ShapeDtypeStruct((L, D // 8, B // 128, 8, 128),
                                      jnp.float32),
        compiler_params=pltpu.CompilerParams(needs_layout_passes=False,
                                             use_tc_tiling_on_sc=False),
        scratch_types=[
            pltpu.VMEM((2, RPC), jnp.int32),
            pltpu.VMEM((2, RPC, D), jnp.float32),
            pltpu.VMEM((L, D), jnp.float32),
            pltpu.VMEM((2, D, TC_), jnp.float32),
            pltpu.VMEM((D,), jnp.float32),
            pltpu.VMEM((D,), jnp.float32),
            pltpu.SemaphoreType.DMA,
            pltpu.SemaphoreType.DMA,
            pltpu.SemaphoreType.DMA,
            pltpu.SemaphoreType.DMA,
        ],
    )(_sc_body)
    out5 = sc(tokT, tok_table, pos_table, gamma, beta)
    return out5.transpose(2, 4, 0, 1, 3).reshape(B, L, D)


# speedup vs baseline: 1.0408x; 1.0408x over previous
import functools

import jax
import jax.numpy as jnp
from jax import lax
from jax.experimental import pallas as pl
from jax.experimental.pallas import tpu as pltpu
from jax.experimental.pallas import tpu_sc as plsc

VOCAB = 1000000
D = 64
B = 4096
L = 200
BL = B * L

NC = 2
NS = 16
NW = NC * NS
BW = B // NW
CH_L = 4
NCH = L // CH_L
RPC = CH_L * BW
NK = D // 16
TC_ = 129


def _rsqrt(x):
    i = plsc.bitcast(x, jnp.int32)
    i = 0x5F3759DF - lax.shift_right_arithmetic(i, 1)
    y = plsc.bitcast(i, jnp.float32)
    for _ in range(2):
        y = y * (1.5 - 0.5 * x * y * y)
    return y


def _splat(v, lane):
    return jnp.take_along_axis(v, jnp.full((16,), lane, jnp.int32), axis=0)


def _sc_body(tokT_hbm, table_hbm, pos_hbm, gb_hbm, bb_hbm, out_hbm,
             idx_v, rows_v, pos_v, tbuf, gb_v, bb_v,
             sem_g0, sem_g1, sem_t0, sem_t1):
    wid = lax.axis_index("s") * NC + lax.axis_index("c")
    pltpu.sync_copy(pos_hbm, pos_v)
    pltpu.sync_copy(gb_hbm, gb_v)
    pltpu.sync_copy(bb_hbm, bb_v)
    gvec = [gb_v[pl.ds(k * 16, 16)] for k in range(NK)]
    bvec = [bb_v[pl.ds(k * 16, 16)] for k in range(NK)]
    lane = lax.iota(jnp.int32, 16)
    didx = [lane + k * 16 for k in range(NK)]
    dhi = [lax.shift_right_logical(dv, 3) for dv in didx]
    dlo = [dv & 7 for dv in didx]
    gsems = [sem_g0, sem_g1]
    tsems = [sem_t0, sem_t1]

    def fetch_idx(buf, ci):
        pltpu.sync_copy(
            tokT_hbm.at[pl.ds(ci * CH_L, CH_L), pl.ds(wid * BW, BW)],
            idx_v.at[buf])

    def fire_gathers(buf, sem):
        for li in range(CH_L):
            pltpu.async_copy(
                table_hbm.at[idx_v.at[buf, li]],
                rows_v.at[buf, pl.ds(li * BW, BW)], sem)

    def drain_gathers(buf, sem):
        for li in range(CH_L):
            pltpu.make_async_copy(
                table_hbm.at[idx_v.at[buf, li]],
                rows_v.at[buf, pl.ds(li * BW, BW)], sem).wait()

    def drain_tiles(tb, lprev):
        pltpu.make_async_copy(
            tbuf.at[tb, pl.ds(0, 8), pl.ds(0, 8), pl.ds(0, 128)],
            out_hbm.at[lprev, pl.ds(0, 8), wid], tsems[tb]).wait()

    def fire_tiles(tb, l):
        pltpu.async_copy(
            tbuf.at[tb, pl.ds(0, 8), pl.ds(0, 8), pl.ds(0, 128)],
            out_hbm.at[l, pl.ds(0, 8), wid], tsems[tb])

    def do_pos(buf, l, li, lofs, tb, guard):
        @pl.when(guard)
        def _():
            drain_tiles(tb, l - 2)

        pk = [pos_v[l, pl.ds(k * 16, 16)] for k in range(NK)]

        def gbody(g, gcarry):
            base = lofs + g * 16
            tok_v = idx_v[buf, li, pl.ds(g * 16, 16)]
            maskf = jnp.where(tok_v != 0, 1.0, 0.0).astype(jnp.float32)
            for r in range(16):
                row = base + r
                s = [rows_v[buf, row, pl.ds(k * 16, 16)] + pk[k]
                     for k in range(NK)]
                part = (s[0] + s[1]) + (s[2] + s[3])
                tot = _splat(plsc.cumsum(part), 15)
                sq = (s[0] * s[0] + s[1] * s[1]) + (s[2] * s[2] + s[3] * s[3])
                tot2 = _splat(plsc.cumsum(sq), 15)
                mu = tot * (1.0 / D)
                var = tot2 * (1.0 / D) - mu * mu + 1e-5
                rs = _rsqrt(var)
                m = _splat(maskf, r)
                a = rs * m
                bco = (0.0 - mu * rs) * m
                col = jnp.full((16,), g * 16 + r, jnp.int32)
                for k in range(NK):
                    o = (s[k] * a + bco) * gvec[k] + bvec[k] * m
                    plsc.store_scatter(tbuf.at[tb], [dhi[k], dlo[k], col], o)
            return gcarry

        lax.fori_loop(0, BW // 16, gbody, 0)
        fire_tiles(tb, l)

    def compute_chunk(buf, ci):
        def lp_body(lp, carry):
            lbase = ci * CH_L + lp * 2
            do_pos(buf, lbase, lp * 2, lp * 2 * BW, 0, ci * 2 + lp > 0)
            do_pos(buf, lbase + 1, lp * 2 + 1, (lp * 2 + 1) * BW, 1,
                   ci * 2 + lp > 0)
            return carry

        lax.fori_loop(0, CH_L // 2, lp_body, 0)

    fetch_idx(0, 0)
    fire_gathers(0, gsems[0])

    def half_body(b, ci):
        nb = 1 - b

        @pl.when(ci < NCH - 1)
        def _prefetch():
            fetch_idx(nb, ci + 1)
            fire_gathers(nb, gsems[nb])

        drain_gathers(b, gsems[b])
        compute_chunk(b, ci)

    def pair_body(cp, carry):
        half_body(0, cp * 2)
        half_body(1, cp * 2 + 1)
        return carry

    lax.fori_loop(0, NCH // 2, pair_body, 0)
    drain_tiles(0, L - 2)
    drain_tiles(1, L - 1)


def kernel(tokens, tok_table, pos_table, gamma, beta):
    tokT = tokens.T.astype(jnp.int32)

    sc = functools.partial(
        pl.kernel,
        mesh=plsc.VectorSubcoreMesh(core_axis_name="c", subcore_axis_name="s"),
        out_type=jax.ShapeDtypeStruct((L, D // 8, B // 128, 8, 128),
                                      jnp.float32),
        compiler_params=pltpu.CompilerParams(needs_layout_passes=False,
                                             use_tc_tiling_on_sc=False),
        scratch_types=[
            pltpu.VMEM((2, CH_L, BW), jnp.int32),
            pltpu.VMEM((2, RPC, D), jnp.float32),
            pltpu.VMEM((L, D), jnp.float32),
            pltpu.VMEM((2, 8, 8, TC_), jnp.float32),
            pltpu.VMEM((D,), jnp.float32),
            pltpu.VMEM((D,), jnp.float32),
            pltpu.SemaphoreType.DMA,
            pltpu.SemaphoreType.DMA,
            pltpu.SemaphoreType.DMA,
            pltpu.SemaphoreType.DMA,
        ],
    )(_sc_body)
    out5 = sc(tokT, tok_table, pos_table, gamma, beta)
    return out5.transpose(2, 4, 0, 1, 3).reshape(B, L, D)
